# baseline (speedup 1.0000x reference)
import functools

import jax
import jax.numpy as jnp
from jax.experimental import pallas as pl

DIM = 1024
HEAD = 16
DK = 64
DV = 64
SPAN = 256
LQ = 2048
BATCH = 2
EPS = 1e-5

TQ = 512
HALO = SPAN
TEXT = TQ + HALO
SUB = 256
NSUB = TQ // SUB
NB = LQ // TQ


def _ln(x, g, b):
    mu = jnp.mean(x, axis=-1, keepdims=True)
    xc = x - mu
    var = jnp.mean(xc * xc, axis=-1, keepdims=True)
    return xc * jax.lax.rsqrt(var + EPS) * g + b


def _block_kernel(q_ref, halo_ref, g_ref, b_ref, wq_ref, wk_ref, wv_ref,
                  wo_ref, w1_ref, b1_ref, w2_ref, b2_ref, out_ref):
    i = pl.program_id(1)
    qs = i * TQ

    g = g_ref[0]
    b = b_ref[0]

    q_blk = q_ref[0]
    x_ext = jnp.concatenate([halo_ref[0], q_blk], axis=0)
    x_ext = _ln(x_ext, g, b)

    scale = jnp.float32(1.0 / (DK ** 0.5))
    x16 = x_ext.astype(jnp.bfloat16)
    qx = jnp.dot(x16[HALO:], wq_ref[...], preferred_element_type=jnp.float32)
    qx = (qx * scale).astype(jnp.bfloat16)
    kx = jnp.dot(x16, wk_ref[...],
                 preferred_element_type=jnp.float32).astype(jnp.bfloat16)
    vx = jnp.dot(x16, wv_ref[...],
                 preferred_element_type=jnp.float32).astype(jnp.bfloat16)

    rows = jax.lax.broadcasted_iota(jnp.int32, (SUB, 2 * SUB), 0)
    cols = jax.lax.broadcasted_iota(jnp.int32, (SUB, 2 * SUB), 1)
    band = (cols >= rows + 1) & (cols <= rows + SPAN)

    sub_outs = []
    for s in range(NSUB):
        kv_base = qs + s * SUB - SPAN
        mask = band & (kv_base + cols >= 0)
        neg = jnp.where(mask, jnp.float32(0), jnp.float32(-1e30))
        head_outs = []
        for h in range(HEAD):
            qh = qx[s * SUB:(s + 1) * SUB, h * DK:(h + 1) * DK]
            kh = kx[s * SUB:s * SUB + 2 * SUB, h * DK:(h + 1) * DK]
            vh = vx[s * SUB:s * SUB + 2 * SUB, h * DV:(h + 1) * DV]
            sc = jax.lax.dot_general(
                qh, kh, (((1,), (1,)), ((), ())),
                preferred_element_type=jnp.float32) + neg
            m = jnp.max(sc, axis=1, keepdims=True)
            e = jnp.exp(sc - m)
            rcp = 1.0 / jnp.sum(e, axis=1, keepdims=True)
            ov = jnp.dot(e.astype(jnp.bfloat16), vh,
                         preferred_element_type=jnp.float32)
            head_outs.append(ov * rcp)
        sub_outs.append(jnp.concatenate(head_outs, axis=1))
    attn = jnp.concatenate(sub_outs, axis=0)

    resid1 = jnp.dot(attn.astype(jnp.bfloat16), wo_ref[...],
                     preferred_element_type=jnp.float32) + q_blk
    x2 = _ln(resid1, g, b)
    h1 = jnp.dot(x2.astype(jnp.bfloat16), w1_ref[...],
                 preferred_element_type=jnp.float32) + b1_ref[0]
    h1 = h1 * 0.5 * (1.0 + jax.lax.erf(h1 * jnp.float32(0.7071067811865476)))
    out = jnp.dot(h1.astype(jnp.bfloat16), w2_ref[...],
                  preferred_element_type=jnp.float32) + b2_ref[0]
    out_ref[0] = out + resid1


@jax.jit
def _run(query, ln_g, ln_b, WqT, WkT, WvT, WoT, W1T, b1, W2T, b2):
    vec = lambda v: v.reshape(1, -1)
    full = lambda arr: pl.BlockSpec(arr.shape, lambda bi, ii: (0,) * arr.ndim)
    grid = (BATCH, NB)
    return pl.pallas_call(
        _block_kernel,
        grid=grid,
        in_specs=[
            pl.BlockSpec((1, TQ, DIM), lambda bi, ii: (bi, ii, 0)),
            pl.BlockSpec((1, HALO, DIM),
                         lambda bi, ii: (bi, jnp.maximum(ii * (TQ // HALO) - 1, 0), 0)),
            full(vec(ln_g)), full(vec(ln_b)),
            full(WqT), full(WkT), full(WvT), full(WoT),
            full(W1T), full(vec(b1)), full(W2T), full(vec(b2)),
        ],
        out_specs=pl.BlockSpec((1, TQ, DIM), lambda bi, ii: (bi, ii, 0)),
        out_shape=jax.ShapeDtypeStruct((BATCH, LQ, DIM), jnp.float32),
    )(query, query, vec(ln_g), vec(ln_b), WqT, WkT, WvT, WoT,
      W1T, vec(b1), W2T, vec(b2))


def kernel(query, ln_g, ln_b, Wq, Wk, Wv, Wo, W1, b1, W2, b2):
    bf = lambda w: w.T.astype(jnp.bfloat16)
    return _run(query, ln_g, ln_b, bf(Wq), bf(Wk), bf(Wv), bf(Wo),
                bf(W1), b1, bf(W2), b2)


# speedup vs baseline: 1.1671x; 1.1671x over previous
import functools

import jax
import jax.numpy as jnp
from jax.experimental import pallas as pl

DIM = 1024
HEAD = 16
DK = 64
DV = 64
SPAN = 256
LQ = 2048
BATCH = 2
EPS = 1e-5

TQ = 512
HALO = SPAN
TEXT = TQ + HALO
SUB = 256
NSUB = TQ // SUB
NB = LQ // TQ


def _ln(x, g, b):
    mu = jnp.mean(x, axis=-1, keepdims=True)
    xc = x - mu
    var = jnp.mean(xc * xc, axis=-1, keepdims=True)
    return xc * jax.lax.rsqrt(var + EPS) * g + b


def _block_kernel(q_ref, halo_ref, g_ref, b_ref, wq_ref, wk_ref, wv_ref,
                  wo_ref, w1_ref, b1_ref, w2_ref, b2_ref, out_ref):
    i = pl.program_id(1)
    qs = i * TQ

    g = g_ref[0]
    b = b_ref[0]

    q_blk = q_ref[0]
    x_ext = jnp.concatenate([halo_ref[0], q_blk], axis=0)
    x_ext = _ln(x_ext, g, b)

    scale = jnp.float32(1.0 / (DK ** 0.5))
    qx = jnp.dot(x_ext[HALO:], wq_ref[...],
                 preferred_element_type=jnp.float32) * scale
    kx = jnp.dot(x_ext, wk_ref[...], preferred_element_type=jnp.float32)
    vx = jnp.dot(x_ext, wv_ref[...], preferred_element_type=jnp.float32)

    rows = jax.lax.broadcasted_iota(jnp.int32, (SUB, 2 * SUB), 0)
    cols = jax.lax.broadcasted_iota(jnp.int32, (SUB, 2 * SUB), 1)
    band = (cols >= rows + 1) & (cols <= rows + SPAN)

    sub_outs = []
    for s in range(NSUB):
        kv_base = qs + s * SUB - SPAN
        mask = band & (kv_base + cols >= 0)
        neg = jnp.where(mask, jnp.float32(0), jnp.float32(-1e30))
        head_outs = []
        for h in range(HEAD):
            qh = qx[s * SUB:(s + 1) * SUB, h * DK:(h + 1) * DK]
            kh = kx[s * SUB:s * SUB + 2 * SUB, h * DK:(h + 1) * DK]
            vh = vx[s * SUB:s * SUB + 2 * SUB, h * DV:(h + 1) * DV]
            sc = jax.lax.dot_general(
                qh, kh, (((1,), (1,)), ((), ())),
                preferred_element_type=jnp.float32) + neg
            m = jnp.max(sc, axis=1, keepdims=True)
            e = jnp.exp(sc - m)
            rcp = 1.0 / jnp.sum(e, axis=1, keepdims=True)
            ov = jnp.dot(e, vh, preferred_element_type=jnp.float32)
            head_outs.append(ov * rcp)
        sub_outs.append(jnp.concatenate(head_outs, axis=1))
    attn = jnp.concatenate(sub_outs, axis=0)

    resid1 = jnp.dot(attn, wo_ref[...],
                     preferred_element_type=jnp.float32) + q_blk
    x2 = _ln(resid1, g, b)
    h1 = jnp.dot(x2, w1_ref[...], preferred_element_type=jnp.float32) + b1_ref[0]
    h1 = h1 * 0.5 * (1.0 + jax.lax.erf(h1 * jnp.float32(0.7071067811865476)))
    out = jnp.dot(h1, w2_ref[...], preferred_element_type=jnp.float32) + b2_ref[0]
    out_ref[0] = out + resid1


@jax.jit
def _run(query, ln_g, ln_b, WqT, WkT, WvT, WoT, W1T, b1, W2T, b2):
    vec = lambda v: v.reshape(1, -1)
    full = lambda arr: pl.BlockSpec(arr.shape, lambda bi, ii: (0,) * arr.ndim)
    grid = (BATCH, NB)
    return pl.pallas_call(
        _block_kernel,
        grid=grid,
        in_specs=[
            pl.BlockSpec((1, TQ, DIM), lambda bi, ii: (bi, ii, 0)),
            pl.BlockSpec((1, HALO, DIM),
                         lambda bi, ii: (bi, jnp.maximum(ii * (TQ // HALO) - 1, 0), 0)),
            full(vec(ln_g)), full(vec(ln_b)),
            full(WqT), full(WkT), full(WvT), full(WoT),
            full(W1T), full(vec(b1)), full(W2T), full(vec(b2)),
        ],
        out_specs=pl.BlockSpec((1, TQ, DIM), lambda bi, ii: (bi, ii, 0)),
        out_shape=jax.ShapeDtypeStruct((BATCH, LQ, DIM), jnp.float32),
    )(query, query, vec(ln_g), vec(ln_b), WqT, WkT, WvT, WoT,
      W1T, vec(b1), W2T, vec(b2))


def kernel(query, ln_g, ln_b, Wq, Wk, Wv, Wo, W1, b1, W2, b2):
    return _run(query, ln_g, ln_b, Wq.T, Wk.T, Wv.T, Wo.T, W1.T, b1, W2.T, b2)


# speedup vs baseline: 1.4788x; 1.2671x over previous
import functools

import jax
import jax.numpy as jnp
from jax.experimental import pallas as pl

DIM = 1024
HEAD = 16
DK = 64
DV = 64
SPAN = 256
LQ = 2048
BATCH = 2
EPS = 1e-5

TQ = 512
HALO = SPAN
TEXT = TQ + HALO
SUB = 256
NSUB = TQ // SUB
NB = LQ // TQ


def _ln(x, g, b):
    mu = jnp.mean(x, axis=-1, keepdims=True)
    xc = x - mu
    var = jnp.mean(xc * xc, axis=-1, keepdims=True)
    return xc * jax.lax.rsqrt(var + EPS) * g + b


def _block_kernel(q_ref, halo_ref, g_ref, b_ref, wq_ref, wk_ref, wv_ref,
                  wo_ref, w1_ref, b1_ref, w2_ref, b2_ref, out_ref):
    i = pl.program_id(1)
    qs = i * TQ

    g = g_ref[0]
    b = b_ref[0]

    q_blk = q_ref[0]
    x_ext = jnp.concatenate([halo_ref[0], q_blk], axis=0)
    x_ext = _ln(x_ext, g, b)

    scale = jnp.float32(1.0 / (DK ** 0.5))
    dnt = (((1,), (1,)), ((), ()))
    qx = jax.lax.dot_general(x_ext[HALO:], wq_ref[...], dnt,
                             preferred_element_type=jnp.float32) * scale
    kx = jax.lax.dot_general(x_ext, wk_ref[...], dnt,
                             preferred_element_type=jnp.float32)
    vx = jax.lax.dot_general(x_ext, wv_ref[...], dnt,
                             preferred_element_type=jnp.float32)

    rows = jax.lax.broadcasted_iota(jnp.int32, (SUB, 2 * SUB), 0)
    cols = jax.lax.broadcasted_iota(jnp.int32, (SUB, 2 * SUB), 1)
    band = (cols >= rows + 1) & (cols <= rows + SPAN)

    sub_outs = []
    for s in range(NSUB):
        kv_base = qs + s * SUB - SPAN
        mask = band & (kv_base + cols >= 0)
        neg = jnp.where(mask, jnp.float32(0), jnp.float32(-1e30))
        head_outs = []
        for h in range(HEAD):
            qh = qx[s * SUB:(s + 1) * SUB, h * DK:(h + 1) * DK]
            kh = kx[s * SUB:s * SUB + 2 * SUB, h * DK:(h + 1) * DK]
            vh = vx[s * SUB:s * SUB + 2 * SUB, h * DV:(h + 1) * DV]
            sc = jax.lax.dot_general(
                qh, kh, (((1,), (1,)), ((), ())),
                preferred_element_type=jnp.float32) + neg
            m = jnp.max(sc, axis=1, keepdims=True)
            e = jnp.exp(sc - m)
            rcp = 1.0 / jnp.sum(e, axis=1, keepdims=True)
            ov = jnp.dot(e, vh, preferred_element_type=jnp.float32)
            head_outs.append(ov * rcp)
        sub_outs.append(jnp.concatenate(head_outs, axis=1))
    attn = jnp.concatenate(sub_outs, axis=0)

    resid1 = jax.lax.dot_general(attn, wo_ref[...], dnt,
                                 preferred_element_type=jnp.float32) + q_blk
    x2 = _ln(resid1, g, b)
    h1 = jax.lax.dot_general(x2, w1_ref[...], dnt,
                             preferred_element_type=jnp.float32) + b1_ref[0]
    h1 = h1 * 0.5 * (1.0 + jax.lax.erf(h1 * jnp.float32(0.7071067811865476)))
    out = jax.lax.dot_general(h1, w2_ref[...], dnt,
                               preferred_element_type=jnp.float32) + b2_ref[0]
    out_ref[0] = out + resid1


@jax.jit
def _run(query, ln_g, ln_b, WqT, WkT, WvT, WoT, W1T, b1, W2T, b2):
    vec = lambda v: v.reshape(1, -1)
    full = lambda arr: pl.BlockSpec(arr.shape, lambda bi, ii: (0,) * arr.ndim)
    grid = (BATCH, NB)
    return pl.pallas_call(
        _block_kernel,
        grid=grid,
        in_specs=[
            pl.BlockSpec((1, TQ, DIM), lambda bi, ii: (bi, ii, 0)),
            pl.BlockSpec((1, HALO, DIM),
                         lambda bi, ii: (bi, jnp.maximum(ii * (TQ // HALO) - 1, 0), 0)),
            full(vec(ln_g)), full(vec(ln_b)),
            full(WqT), full(WkT), full(WvT), full(WoT),
            full(W1T), full(vec(b1)), full(W2T), full(vec(b2)),
        ],
        out_specs=pl.BlockSpec((1, TQ, DIM), lambda bi, ii: (bi, ii, 0)),
        out_shape=jax.ShapeDtypeStruct((BATCH, LQ, DIM), jnp.float32),
    )(query, query, vec(ln_g), vec(ln_b), WqT, WkT, WvT, WoT,
      W1T, vec(b1), W2T, vec(b2))


def kernel(query, ln_g, ln_b, Wq, Wk, Wv, Wo, W1, b1, W2, b2):
    return _run(query, ln_g, ln_b, Wq, Wk, Wv, Wo, W1, b1, W2, b2)


# speedup vs baseline: 1.4961x; 1.0117x over previous
import functools

import jax
import jax.numpy as jnp
from jax.experimental import pallas as pl

DIM = 1024
HEAD = 16
DK = 64
DV = 64
SPAN = 256
LQ = 2048
BATCH = 2
EPS = 1e-5

TQ = 512
HALO = SPAN
TEXT = TQ + HALO
SUB = 256
NSUB = TQ // SUB
NB = LQ // TQ


def _ln(x, g, b):
    mu = jnp.mean(x, axis=-1, keepdims=True)
    xc = x - mu
    var = jnp.mean(xc * xc, axis=-1, keepdims=True)
    return xc * jax.lax.rsqrt(var + EPS) * g + b


def _block_kernel(q_ref, halo_ref, g_ref, b_ref, wq_ref, wk_ref, wv_ref,
                  wo_ref, w1_ref, b1_ref, w2_ref, b2_ref, out_ref):
    i = pl.program_id(1)
    qs = i * TQ

    g = g_ref[0]
    b = b_ref[0]

    q_blk = q_ref[0]
    x_ext = jnp.concatenate([halo_ref[0], q_blk], axis=0)
    x_ext = _ln(x_ext, g, b)

    scale = jnp.float32(1.4426950408889634 / (DK ** 0.5))
    dnt = (((1,), (1,)), ((), ()))
    qx = jax.lax.dot_general(x_ext[HALO:], wq_ref[...], dnt,
                             preferred_element_type=jnp.float32) * scale
    kx = jax.lax.dot_general(x_ext, wk_ref[...], dnt,
                             preferred_element_type=jnp.float32)
    vx = jax.lax.dot_general(x_ext, wv_ref[...], dnt,
                             preferred_element_type=jnp.float32)

    rows = jax.lax.broadcasted_iota(jnp.int32, (SUB, 2 * SUB), 0)
    cols = jax.lax.broadcasted_iota(jnp.int32, (SUB, 2 * SUB), 1)
    band = (cols >= rows + 1) & (cols <= rows + SPAN)

    sub_outs = []
    for s in range(NSUB):
        kv_base = qs + s * SUB - SPAN
        mask = band & (kv_base + cols >= 0)
        neg = jnp.where(mask, jnp.float32(0), jnp.float32(-1e30))
        head_outs = []
        for h in range(HEAD):
            qh = qx[s * SUB:(s + 1) * SUB, h * DK:(h + 1) * DK]
            kh = kx[s * SUB:s * SUB + 2 * SUB, h * DK:(h + 1) * DK]
            vh = vx[s * SUB:s * SUB + 2 * SUB, h * DV:(h + 1) * DV]
            sc = jax.lax.dot_general(
                qh, kh, (((1,), (1,)), ((), ())),
                preferred_element_type=jnp.float32) + neg
            m = jnp.max(sc, axis=1, keepdims=True)
            e = jnp.exp2(sc - m)
            rcp = 1.0 / jnp.sum(e, axis=1, keepdims=True)
            ov = jnp.dot(e, vh, preferred_element_type=jnp.float32)
            head_outs.append(ov * rcp)
        sub_outs.append(jnp.concatenate(head_outs, axis=1))
    attn = jnp.concatenate(sub_outs, axis=0)

    resid1 = jax.lax.dot_general(attn, wo_ref[...], dnt,
                                 preferred_element_type=jnp.float32) + q_blk
    x2 = _ln(resid1, g, b)
    h1 = jax.lax.dot_general(x2, w1_ref[...], dnt,
                             preferred_element_type=jnp.float32) + b1_ref[0]
    h1 = h1 * 0.5 * (1.0 + jax.lax.erf(h1 * jnp.float32(0.7071067811865476)))
    out = jax.lax.dot_general(h1, w2_ref[...], dnt,
                               preferred_element_type=jnp.float32) + b2_ref[0]
    out_ref[0] = out + resid1


@jax.jit
def _run(query, ln_g, ln_b, WqT, WkT, WvT, WoT, W1T, b1, W2T, b2):
    vec = lambda v: v.reshape(1, -1)
    full = lambda arr: pl.BlockSpec(arr.shape, lambda bi, ii: (0,) * arr.ndim)
    grid = (BATCH, NB)
    return pl.pallas_call(
        _block_kernel,
        grid=grid,
        in_specs=[
            pl.BlockSpec((1, TQ, DIM), lambda bi, ii: (bi, ii, 0)),
            pl.BlockSpec((1, HALO, DIM),
                         lambda bi, ii: (bi, jnp.maximum(ii * (TQ // HALO) - 1, 0), 0)),
            full(vec(ln_g)), full(vec(ln_b)),
            full(WqT), full(WkT), full(WvT), full(WoT),
            full(W1T), full(vec(b1)), full(W2T), full(vec(b2)),
        ],
        out_specs=pl.BlockSpec((1, TQ, DIM), lambda bi, ii: (bi, ii, 0)),
        out_shape=jax.ShapeDtypeStruct((BATCH, LQ, DIM), jnp.float32),
    )(query, query, vec(ln_g), vec(ln_b), WqT, WkT, WvT, WoT,
      W1T, vec(b1), W2T, vec(b2))


def kernel(query, ln_g, ln_b, Wq, Wk, Wv, Wo, W1, b1, W2, b2):
    return _run(query, ln_g, ln_b, Wq, Wk, Wv, Wo, W1, b1, W2, b2)


# speedup vs baseline: 1.4973x; 1.0008x over previous
import jax
import jax.numpy as jnp
from jax.experimental import pallas as pl

DIM = 1024
HEAD = 16
DK = 64
DV = 64
SPAN = 256
LQ = 2048
BATCH = 2
EPS = 1e-5

TQ = 512
HALO = SPAN
TEXT = TQ + HALO
SUB = 256
NSUB = TQ // SUB
NB = LQ // TQ


def _ln(x, g, b):
    mu = jnp.mean(x, axis=-1, keepdims=True)
    xc = x - mu
    var = jnp.mean(xc * xc, axis=-1, keepdims=True)
    return xc * jax.lax.rsqrt(var + EPS) * g + b


def _block_kernel(q_ref, halo_ref, g_ref, b_ref, wq_ref, wk_ref, wv_ref,
                  wo_ref, w1_ref, b1_ref, w2_ref, b2_ref, out_ref):
    i = pl.program_id(1)
    qs = i * TQ

    g = g_ref[0]
    b = b_ref[0]

    q_blk = q_ref[0]
    x_ext = jnp.concatenate([halo_ref[0], q_blk], axis=0)
    x_ext = _ln(x_ext, g, b)

    scale = jnp.float32(1.4426950408889634 / (DK ** 0.5))
    dnt = (((1,), (1,)), ((), ()))
    qx = jax.lax.dot_general(x_ext[HALO:], wq_ref[...], dnt,
                             preferred_element_type=jnp.float32) * scale
    kx = jax.lax.dot_general(x_ext, wk_ref[...], dnt,
                             preferred_element_type=jnp.float32)
    vx = jax.lax.dot_general(x_ext, wv_ref[...], dnt,
                             preferred_element_type=jnp.float32)

    rows = jax.lax.broadcasted_iota(jnp.int32, (SUB, 2 * SUB), 0)
    cols = jax.lax.broadcasted_iota(jnp.int32, (SUB, 2 * SUB), 1)
    band = (cols >= rows + 1) & (cols <= rows + SPAN)

    sub_outs = []
    for s in range(NSUB):
        kv_base = qs + s * SUB - SPAN
        mask = band & (kv_base + cols >= 0)
        neg = jnp.where(mask, jnp.float32(0), jnp.float32(-1e30))
        head_outs = []
        for h in range(HEAD):
            qh = qx[s * SUB:(s + 1) * SUB, h * DK:(h + 1) * DK]
            kh = kx[s * SUB:s * SUB + 2 * SUB, h * DK:(h + 1) * DK]
            vh = vx[s * SUB:s * SUB + 2 * SUB, h * DV:(h + 1) * DV]
            sc = jax.lax.dot_general(
                qh, kh, (((1,), (1,)), ((), ())),
                preferred_element_type=jnp.float32) + neg
            m = jnp.max(sc, axis=1, keepdims=True)
            e = jnp.exp2(sc - m)
            rcp = 1.0 / jnp.sum(e, axis=1, keepdims=True)
            ov = jnp.dot(e, vh, preferred_element_type=jnp.float32)
            head_outs.append(ov * rcp)
        sub_outs.append(jnp.concatenate(head_outs, axis=1))
    attn = jnp.concatenate(sub_outs, axis=0)

    resid1 = jax.lax.dot_general(attn, wo_ref[...], dnt,
                                 preferred_element_type=jnp.float32) + q_blk
    x2 = _ln(resid1, g, b)
    h1 = jax.lax.dot_general(x2, w1_ref[...], dnt,
                             preferred_element_type=jnp.float32) + b1_ref[0]
    h1 = h1 * 0.5 * (1.0 + jax.lax.erf(h1 * jnp.float32(0.7071067811865476)))
    out = jax.lax.dot_general(h1, w2_ref[...], dnt,
                               preferred_element_type=jnp.float32) + b2_ref[0]
    out_ref[0] = out + resid1


@jax.jit
def _run(query, ln_g, ln_b, Wq, Wk, Wv, Wo, W1, b1, W2, b2):
    vec = lambda v: v.reshape(1, -1)
    full = lambda arr: pl.BlockSpec(arr.shape, lambda bi, ii: (0,) * arr.ndim)
    grid = (BATCH, NB)
    return pl.pallas_call(
        _block_kernel,
        grid=grid,
        in_specs=[
            pl.BlockSpec((1, TQ, DIM), lambda bi, ii: (bi, ii, 0)),
            pl.BlockSpec((1, HALO, DIM),
                         lambda bi, ii: (bi, jnp.maximum(ii * (TQ // HALO) - 1, 0), 0)),
            full(vec(ln_g)), full(vec(ln_b)),
            full(Wq), full(Wk), full(Wv), full(Wo),
            full(W1), full(vec(b1)), full(W2), full(vec(b2)),
        ],
        out_specs=pl.BlockSpec((1, TQ, DIM), lambda bi, ii: (bi, ii, 0)),
        out_shape=jax.ShapeDtypeStruct((BATCH, LQ, DIM), jnp.float32),
    )(query, query, vec(ln_g), vec(ln_b), Wq, Wk, Wv, Wo,
      W1, vec(b1), W2, vec(b2))


def kernel(query, ln_g, ln_b, Wq, Wk, Wv, Wo, W1, b1, W2, b2):
    return _run(query, ln_g, ln_b, Wq, Wk, Wv, Wo, W1, b1, W2, b2)
